# baseline (speedup 1.0000x reference)
import functools

import jax
import jax.numpy as jnp
from jax import lax
from jax.experimental import pallas as pl
from jax.experimental.pallas import tpu as pltpu
from jax.experimental.pallas import tpu_sc as plsc

NC = 2
NS = 16
NW = NC * NS
L = 16

N = 10000
B = 320000
D = 128
E = B // NW
C = 80
NCHUNK = E // C
K = D // L
NBUF = 6
PROMO = 3


def _dot_decode_body(z_hbm, src_hbm, dst_hbm, n_hbm, out_hbm,
                     idx_s, idx_d, out_v, n_v,
                     b0, b1, b2, b3, b4, b5,
                     sem0, sem1, sem2, sem3, sem4, sem5):
    wid = lax.axis_index("s") * NC + lax.axis_index("c")
    base = pl.multiple_of(wid * E, 8)

    pltpu.sync_copy(src_hbm.at[pl.ds(base, E)], idx_s)
    pltpu.sync_copy(dst_hbm.at[pl.ds(base, E)], idx_d)
    pltpu.sync_copy(n_hbm, n_v)

    bufs = ((b0, sem0), (b1, sem1), (b2, sem2),
            (b3, sem3), (b4, sem4), (b5, sem5))

    def issue_src(c, b):
        buf, sem = bufs[b]
        off = pl.multiple_of(c * C, 8)
        pltpu.async_copy(z_hbm.at[idx_s.at[pl.ds(off, C)]], buf, sem)

    def issue_add(c, b):
        buf, sem = bufs[b]
        off = pl.multiple_of(c * C, 8)
        pltpu.async_copy(z_hbm.at[idx_d.at[pl.ds(off, C)]], buf, sem,
                         add=True)

    def wait_stream(c, b):
        buf, sem = bufs[b]
        off = pl.multiple_of(c * C, 8)
        pltpu.make_async_copy(z_hbm.at[idx_s.at[pl.ds(off, C)]], buf,
                              sem).wait()

    lane = lax.iota(jnp.int32, L)
    lane15 = lane == (L - 1)

    def compute(c, b):
        buf, _ = bufs[b]

        def group_body(g, _):
            gbase = c * C + g * L
            for e in range(L):
                row = g * L + e
                v = buf[row, pl.ds(0, L)]
                acc = v * v
                for k in range(1, K):
                    v = buf[row, pl.ds(k * L, L)]
                    acc = acc + v * v
                cum = plsc.cumsum(acc)
                plsc.store_scatter(
                    out_v, [jnp.full((L,), gbase + e, jnp.int32)],
                    cum, mask=lane15)
            return 0

        lax.fori_loop(0, C // L, group_body, 0)

    for b in range(NBUF):
        issue_src(b, b)
    for c in range(PROMO):
        wait_stream(c, c)
        issue_add(c, c)

    def outer(i, _):
        for b in range(NBUF):
            c = NBUF * i + b

            @pl.when(c < NCHUNK)
            def _():
                bp = (b + PROMO) % NBUF

                @pl.when(c + PROMO < NCHUNK)
                def _():
                    wait_stream(c + PROMO, bp)
                    issue_add(c + PROMO, bp)

                wait_stream(c, b)
                compute(c, b)

                @pl.when(c + NBUF < NCHUNK)
                def _():
                    issue_src(c + NBUF, b)
        return 0

    lax.fori_loop(0, (NCHUNK + NBUF - 1) // NBUF, outer, 0)

    def fin_body(g, _):
        off = pl.multiple_of(g * L, 8)
        ssq = out_v[pl.ds(off, L)]
        si = idx_s[pl.ds(off, L)]
        di = idx_d[pl.ds(off, L)]
        ns = plsc.load_gather(n_v, [si])
        nd = plsc.load_gather(n_v, [di])
        val = 0.5 * ssq - 0.5 * (ns + nd)
        out_v[pl.ds(off, L)] = 1.0 / (1.0 + jnp.exp(-val))
        return 0

    lax.fori_loop(0, E // L, fin_body, 0, unroll=2)
    pltpu.sync_copy(out_v, out_hbm.at[pl.ds(base, E)])


def _row_sqnorm_body(z_ref, n_ref):
    z = z_ref[...]
    n_ref[...] = jnp.sum(z * z, axis=1, keepdims=True)


@jax.jit
def kernel(z, edge_index):
    sqn = pl.pallas_call(
        _row_sqnorm_body,
        out_shape=jax.ShapeDtypeStruct((N, 1), jnp.float32),
    )(z)
    sqn = sqn.reshape((N,))

    mesh = plsc.VectorSubcoreMesh(core_axis_name="c", subcore_axis_name="s")
    f = pl.kernel(
        _dot_decode_body,
        out_type=jax.ShapeDtypeStruct((B,), jnp.float32),
        mesh=mesh,
        compiler_params=pltpu.CompilerParams(needs_layout_passes=False),
        scratch_types=[
            pltpu.VMEM((E,), jnp.int32),
            pltpu.VMEM((E,), jnp.int32),
            pltpu.VMEM((E,), jnp.float32),
            pltpu.VMEM((N,), jnp.float32),
            *([pltpu.VMEM((C, D), jnp.float32)] * NBUF),
            *([pltpu.SemaphoreType.DMA] * NBUF),
        ],
    )
    return f(z, edge_index[0], edge_index[1], sqn)


# speedup vs baseline: 1.0905x; 1.0905x over previous
import functools

import jax
import jax.numpy as jnp
from jax import lax
from jax.experimental import pallas as pl
from jax.experimental.pallas import tpu as pltpu
from jax.experimental.pallas import tpu_sc as plsc

NC = 2
NS = 16
NW = NC * NS
L = 16

B = 320000
D = 128
E = B // NW
C = 128
NCHUNK = E // C
TAIL = E - NCHUNK * C
K = D // L
NBUF = 3


def _dot_decode_body(z_hbm, src_hbm, dst_hbm, out_hbm,
                     idx_s, idx_d, out_v,
                     rs0, rd0, rs1, rd1, rs2, rd2,
                     sem0, sem1, sem2):
    wid = lax.axis_index("s") * NC + lax.axis_index("c")
    base = pl.multiple_of(wid * E, 8)

    pltpu.sync_copy(src_hbm.at[pl.ds(base, E)], idx_s)
    pltpu.sync_copy(dst_hbm.at[pl.ds(base, E)], idx_d)

    bufs = ((rs0, rd0, sem0), (rs1, rd1, sem1), (rs2, rd2, sem2))

    def issue(c, b, n=C):
        rs, rd, sem = bufs[b]
        off = pl.multiple_of(c * C, 8)
        pltpu.async_copy(z_hbm.at[idx_s.at[pl.ds(off, n)]],
                         rs.at[pl.ds(0, n)], sem)
        pltpu.async_copy(z_hbm.at[idx_d.at[pl.ds(off, n)]],
                         rd.at[pl.ds(0, n)], sem)

    def wait(c, b, n=C):
        rs, rd, sem = bufs[b]
        off = pl.multiple_of(c * C, 8)
        pltpu.make_async_copy(z_hbm.at[idx_s.at[pl.ds(off, n)]],
                              rs.at[pl.ds(0, n)], sem).wait()
        pltpu.make_async_copy(z_hbm.at[idx_d.at[pl.ds(off, n)]],
                              rd.at[pl.ds(0, n)], sem).wait()

    lane = lax.iota(jnp.int32, L)
    lane15 = lane == (L - 1)

    def compute_group(rs, rd, g, gbase):
        for e in range(L):
            row = g * L + e
            acc = rs[row, pl.ds(0, L)] * rd[row, pl.ds(0, L)]
            for k in range(1, K):
                acc = acc + rs[row, pl.ds(k * L, L)] * rd[row, pl.ds(k * L, L)]
            cum = plsc.cumsum(acc)
            plsc.store_scatter(
                out_v, [jnp.full((L,), gbase + e, jnp.int32)],
                cum, mask=lane15)

    def compute(c, b):
        rs, rd, _ = bufs[b]

        def group_body(g, _):
            compute_group(rs, rd, g, c * C + g * L)
            return 0

        lax.fori_loop(0, C // L, group_body, 0)

    for b in range(NBUF):
        issue(b, b)

    def outer(i, _):
        for b in range(NBUF):
            c = NBUF * i + b
            wait(c, b)
            compute(c, b)

            @pl.when(c + NBUF < NCHUNK)
            def _():
                issue(c + NBUF, b)
        return 0

    lax.fori_loop(0, NCHUNK // NBUF, outer, 0)

    issue(NCHUNK, 0, n=TAIL)
    wait(NCHUNK, 0, n=TAIL)
    rs, rd, _ = bufs[0]
    for g in range(TAIL // L):
        compute_group(rs, rd, g, NCHUNK * C + g * L)

    def sig_body(g, _):
        off = pl.multiple_of(g * L, 8)
        v = out_v[pl.ds(off, L)]
        out_v[pl.ds(off, L)] = 1.0 / (1.0 + jnp.exp(-v))
        return 0

    lax.fori_loop(0, E // L, sig_body, 0, unroll=2)
    pltpu.sync_copy(out_v, out_hbm.at[pl.ds(base, E)])


@jax.jit
def kernel(z, edge_index):
    mesh = plsc.VectorSubcoreMesh(core_axis_name="c", subcore_axis_name="s")
    f = pl.kernel(
        _dot_decode_body,
        out_type=jax.ShapeDtypeStruct((B,), jnp.float32),
        mesh=mesh,
        compiler_params=pltpu.CompilerParams(needs_layout_passes=False),
        scratch_types=[
            pltpu.VMEM((E,), jnp.int32),
            pltpu.VMEM((E,), jnp.int32),
            pltpu.VMEM((E,), jnp.float32),
            *([pltpu.VMEM((C, D), jnp.float32)] * (2 * NBUF)),
            *([pltpu.SemaphoreType.DMA] * NBUF),
        ],
    )
    return f(z, edge_index[0], edge_index[1])


# speedup vs baseline: 1.6976x; 1.5567x over previous
import functools

import jax
import jax.numpy as jnp
from jax import lax
from jax.experimental import pallas as pl
from jax.experimental.pallas import tpu as pltpu
from jax.experimental.pallas import tpu_sc as plsc

NC = 2
NS = 16
NW = NC * NS
L = 16

B = 320000
D = 128
E = B // NW
C = 128
NCHUNK = E // C
TAIL = E - NCHUNK * C
K = D // L
NBUF = 3


def _dot_decode_body(z_hbm, src_hbm, dst_hbm, out_hbm,
                     idx_s, idx_d, out_v,
                     rs0, rd0, rs1, rd1, rs2, rd2,
                     sem0, sem1, sem2):
    wid = lax.axis_index("s") * NC + lax.axis_index("c")
    base = pl.multiple_of(wid * E, 8)

    pltpu.sync_copy(src_hbm.at[pl.ds(base, E)], idx_s)
    pltpu.sync_copy(dst_hbm.at[pl.ds(base, E)], idx_d)

    bufs = ((rs0, rd0, sem0), (rs1, rd1, sem1), (rs2, rd2, sem2))

    def issue(c, b, n=C):
        rs, rd, sem = bufs[b]
        off = pl.multiple_of(c * C, 8)
        pltpu.async_copy(z_hbm.at[idx_s.at[pl.ds(off, n)]],
                         rs.at[pl.ds(0, n)], sem)
        pltpu.async_copy(z_hbm.at[idx_d.at[pl.ds(off, n)]],
                         rd.at[pl.ds(0, n)], sem)

    def wait(c, b, n=C):
        rs, rd, sem = bufs[b]
        off = pl.multiple_of(c * C, 8)
        pltpu.make_async_copy(z_hbm.at[idx_s.at[pl.ds(off, n)]],
                              rs.at[pl.ds(0, n)], sem).wait()
        pltpu.make_async_copy(z_hbm.at[idx_d.at[pl.ds(off, n)]],
                              rd.at[pl.ds(0, n)], sem).wait()

    lane = lax.iota(jnp.int32, L)
    lane15 = lane == (L - 1)

    def edge_body(rs, rd, row, obase):
        prods = [rs[row, pl.ds(k * L, L)] * rd[row, pl.ds(k * L, L)]
                 for k in range(K)]
        while len(prods) > 1:
            prods = [a + b for a, b in zip(prods[0::2], prods[1::2])]
        cum = plsc.cumsum(prods[0])
        plsc.store_compressed(out_v.at[pl.ds(obase, L)], cum, mask=lane15)

    def compute(c, b, n=C):
        rs, rd, _ = bufs[b]

        @plsc.parallel_loop(0, n, unroll=4)
        def _(e):
            edge_body(rs, rd, e, c * C + e)

    for b in range(NBUF):
        issue(b, b)

    def outer(i, _):
        for b in range(NBUF):
            c = NBUF * i + b
            wait(c, b)
            compute(c, b)

            @pl.when(c + NBUF < NCHUNK)
            def _():
                issue(c + NBUF, b)
        return 0

    lax.fori_loop(0, NCHUNK // NBUF, outer, 0)

    issue(NCHUNK, 0, n=TAIL)
    wait(NCHUNK, 0, n=TAIL)
    compute(NCHUNK, 0, n=TAIL)

    def sig_body(g, _):
        off = pl.multiple_of(g * L, 8)
        v = out_v[pl.ds(off, L)]
        out_v[pl.ds(off, L)] = 1.0 / (1.0 + jnp.exp(-v))
        return 0

    lax.fori_loop(0, E // L, sig_body, 0, unroll=2)
    pltpu.sync_copy(out_v.at[pl.ds(0, E)], out_hbm.at[pl.ds(base, E)])


@jax.jit
def kernel(z, edge_index):
    mesh = plsc.VectorSubcoreMesh(core_axis_name="c", subcore_axis_name="s")
    f = pl.kernel(
        _dot_decode_body,
        out_type=jax.ShapeDtypeStruct((B,), jnp.float32),
        mesh=mesh,
        compiler_params=pltpu.CompilerParams(needs_layout_passes=False),
        scratch_types=[
            pltpu.VMEM((E,), jnp.int32),
            pltpu.VMEM((E,), jnp.int32),
            pltpu.VMEM((E + L,), jnp.float32),
            *([pltpu.VMEM((C, D), jnp.float32)] * (2 * NBUF)),
            *([pltpu.SemaphoreType.DMA] * NBUF),
        ],
    )
    return f(z, edge_index[0], edge_index[1])
